# baseline (speedup 1.0000x reference)
import functools

import jax
import jax.numpy as jnp
from jax import lax
from jax.experimental import pallas as pl
from jax.experimental.pallas import tpu as pltpu
from jax.experimental.pallas import tpu_sc as plsc

L = 16
NC, NS = 2, 16
NW = NC * NS


def _make_kernel(n_rays, n_samples, chunk, ray_unroll):
    rays_per_w = n_rays // NW
    n_chunks = rays_per_w // chunk
    nv = n_samples // L
    ns = n_samples
    f32 = jnp.float32

    def body(sig_h, rgb_h, z_h, dst_h, col_h, dep_h, alp_h, w_h,
             sig_v, rgb_v, z_v, dst_v, w_v, col_v, dep_v, alp_v):
        cid = lax.axis_index("c")
        sid = lax.axis_index("s")
        wid = sid * NC + cid
        base_w = wid * rays_per_w
        iota = lax.iota(jnp.int32, L)
        iota3 = iota * 3
        lane0 = iota == 0

        def put1(ref, addr, val):
            plsc.store_scatter(
                ref, [jnp.broadcast_to(addr, (L,)).astype(jnp.int32)],
                jnp.broadcast_to(val, (L,)), mask=lane0)

        def do_ray(r):
            sig = [sig_v[r, pl.ds(j * L, L)] for j in range(nv)]
            dst = [dst_v[r, pl.ds(j * L, L)] for j in range(nv)]
            tau = [jnp.maximum(sig[j], 0.0) * dst[j] for j in range(nv)]
            tot = [jnp.sum(tau[j]) for j in range(nv)]
            c = [jnp.float32(0.0)]
            for j in range(nv):
                c.append(c[j] + tot[j])
            scan = [plsc.cumsum(tau[j]) for j in range(nv)]
            dep_a = jnp.zeros((L,), f32)
            alp_a = jnp.zeros((L,), f32)
            cr_a = jnp.zeros((L,), f32)
            cg_a = jnp.zeros((L,), f32)
            cb_a = jnp.zeros((L,), f32)
            E_last = None
            rsplat = jnp.broadcast_to(r, (L,)).astype(jnp.int32)
            for j in range(nv):
                S = scan[j] + c[j]
                E = jnp.exp(-S)
                Ep = jnp.exp(tau[j] - S)
                w = Ep - E
                w_v[r, pl.ds(j * L, L)] = w
                zz = z_v[r, pl.ds(j * L, L)]
                idx = j * 3 * L + iota3
                rc = plsc.load_gather(rgb_v, [rsplat, idx])
                gc = plsc.load_gather(rgb_v, [rsplat, idx + 1])
                bc = plsc.load_gather(rgb_v, [rsplat, idx + 2])
                dep_a = dep_a + w * zz
                alp_a = alp_a + w
                cr_a = cr_a + w * rc
                cg_a = cg_a + w * gc
                cb_a = cb_a + w * bc
                E_last = E
            no_hit = jnp.min(E_last)
            put1(col_v, 3 * r, jnp.sum(cr_a) + no_hit)
            put1(col_v, 3 * r + 1, jnp.sum(cg_a) + no_hit)
            put1(col_v, 3 * r + 2, jnp.sum(cb_a) + no_hit)
            put1(dep_v, r, jnp.sum(dep_a))
            put1(alp_v, r, jnp.sum(alp_a))

        def chunk_body(k, carry):
            base = base_w + k * chunk
            pltpu.sync_copy(sig_h.at[pl.ds(base, chunk)], sig_v)
            pltpu.sync_copy(rgb_h.at[pl.ds(base, chunk)], rgb_v)
            pltpu.sync_copy(z_h.at[pl.ds(base, chunk)], z_v)
            pltpu.sync_copy(dst_h.at[pl.ds(base, chunk)], dst_v)

            def ray_body(rr, c2):
                for u in range(ray_unroll):
                    do_ray(rr * ray_unroll + u)
                return c2

            lax.fori_loop(0, chunk // ray_unroll, ray_body, 0)

            pltpu.sync_copy(w_v, w_h.at[pl.ds(base, chunk)])
            pltpu.sync_copy(col_v, col_h.at[pl.ds(base * 3, chunk * 3)])
            pltpu.sync_copy(dep_v, dep_h.at[pl.ds(base, chunk)])
            pltpu.sync_copy(alp_v, alp_h.at[pl.ds(base, chunk)])
            return carry

        lax.fori_loop(0, n_chunks, chunk_body, 0)

    mesh = plsc.VectorSubcoreMesh(core_axis_name="c", subcore_axis_name="s")
    return pl.kernel(
        body,
        out_type=(
            jax.ShapeDtypeStruct((n_rays * 3,), f32),
            jax.ShapeDtypeStruct((n_rays,), f32),
            jax.ShapeDtypeStruct((n_rays,), f32),
            jax.ShapeDtypeStruct((n_rays, n_samples), f32),
        ),
        mesh=mesh,
        compiler_params=pltpu.CompilerParams(needs_layout_passes=False),
        scratch_types=[
            pltpu.VMEM((chunk, n_samples), f32),
            pltpu.VMEM((chunk, 3 * n_samples), f32),
            pltpu.VMEM((chunk, n_samples), f32),
            pltpu.VMEM((chunk, n_samples), f32),
            pltpu.VMEM((chunk, n_samples), f32),
            pltpu.VMEM((chunk * 3,), f32),
            pltpu.VMEM((chunk,), f32),
            pltpu.VMEM((chunk,), f32),
        ],
    )


@functools.partial(jax.jit, static_argnums=())
def kernel(sigma_vals, rgb_vals, z_vals, dists):
    n_rays, n_samples = sigma_vals.shape
    k = _make_kernel(n_rays, n_samples, chunk=64, ray_unroll=2)
    color, depth, alpha_coarse, weights = k(
        sigma_vals,
        rgb_vals.reshape(n_rays, 3 * n_samples),
        z_vals,
        dists,
    )
    return color.reshape(n_rays, 3), depth, alpha_coarse, weights


# speedup vs baseline: 1.2902x; 1.2902x over previous
import functools

import jax
import jax.numpy as jnp
from jax import lax
from jax.experimental import pallas as pl
from jax.experimental.pallas import tpu as pltpu
from jax.experimental.pallas import tpu_sc as plsc

L = 16
NC, NS = 2, 16
NW = NC * NS


def _make_kernel(n_rays, n_samples, chunk, ray_unroll):
    rays_per_w = n_rays // NW
    n_chunks = rays_per_w // chunk
    assert n_chunks % 2 == 0
    nv = n_samples // L
    ns = n_samples
    col_rows = rays_per_w * 3 // 128
    sca_rows = rays_per_w // 128
    f32 = jnp.float32

    def body(sig_h, rgb_h, z_h, dst_h, col_h, dep_h, alp_h, w_h,
             sig_v, rgb_v, z_v, dst_v, w_v, col_s, dep_s, alp_s,
             sem_in, sem_out):
        cid = lax.axis_index("c")
        sid = lax.axis_index("s")
        wid = sid * NC + cid
        base_w = wid * rays_per_w
        iota = lax.iota(jnp.int32, L)
        iota3 = iota * 3
        lane0 = iota == 0

        def in_copies(k, s):
            base = base_w + k * chunk
            return [
                pltpu.make_async_copy(sig_h.at[pl.ds(base, chunk)],
                                      sig_v.at[s], sem_in.at[s]),
                pltpu.make_async_copy(rgb_h.at[pl.ds(base, chunk)],
                                      rgb_v.at[s], sem_in.at[s]),
                pltpu.make_async_copy(z_h.at[pl.ds(base, chunk)],
                                      z_v.at[s], sem_in.at[s]),
                pltpu.make_async_copy(dst_h.at[pl.ds(base, chunk)],
                                      dst_v.at[s], sem_in.at[s]),
            ]

        def out_copy(k, s):
            base = base_w + k * chunk
            return pltpu.make_async_copy(
                w_v.at[s], w_h.at[pl.ds(base, chunk)], sem_out.at[s])

        def put1(ref, fi, val):
            row = jnp.broadcast_to(fi >> 7, (L,)).astype(jnp.int32)
            colm = jnp.broadcast_to(fi & 127, (L,)).astype(jnp.int32)
            plsc.store_scatter(ref, [row, colm],
                               jnp.broadcast_to(val, (L,)), mask=lane0)

        def do_ray(k, s, r):
            sig = [sig_v[s, r, pl.ds(j * L, L)] for j in range(nv)]
            dst = [dst_v[s, r, pl.ds(j * L, L)] for j in range(nv)]
            tau = [jnp.maximum(sig[j], 0.0) * dst[j] for j in range(nv)]
            tot = [jnp.sum(tau[j]) for j in range(nv)]
            c = [jnp.float32(0.0)]
            for j in range(nv):
                c.append(c[j] + tot[j])
            scan = [plsc.cumsum(tau[j]) for j in range(nv)]
            dep_a = jnp.zeros((L,), f32)
            alp_a = jnp.zeros((L,), f32)
            cr_a = jnp.zeros((L,), f32)
            cg_a = jnp.zeros((L,), f32)
            cb_a = jnp.zeros((L,), f32)
            E_last = None
            rsplat = jnp.broadcast_to(r, (L,)).astype(jnp.int32)
            for j in range(nv):
                S = scan[j] + c[j]
                E = jnp.exp(-S)
                Ep = jnp.exp(tau[j] - S)
                w = Ep - E
                w_v[s, r, pl.ds(j * L, L)] = w
                zz = z_v[s, r, pl.ds(j * L, L)]
                idx = j * 3 * L + iota3
                rc = plsc.load_gather(rgb_v.at[s], [rsplat, idx])
                gc = plsc.load_gather(rgb_v.at[s], [rsplat, idx + 1])
                bc = plsc.load_gather(rgb_v.at[s], [rsplat, idx + 2])
                dep_a = dep_a + w * zz
                alp_a = alp_a + w
                cr_a = cr_a + w * rc
                cg_a = cg_a + w * gc
                cb_a = cb_a + w * bc
                E_last = E
            no_hit = jnp.min(E_last)
            gray = k * chunk + r
            put1(col_s, 3 * gray, jnp.sum(cr_a) + no_hit)
            put1(col_s, 3 * gray + 1, jnp.sum(cg_a) + no_hit)
            put1(col_s, 3 * gray + 2, jnp.sum(cb_a) + no_hit)
            put1(dep_s, gray, jnp.sum(dep_a))
            put1(alp_s, gray, jnp.sum(alp_a))

        def compute_chunk(k, s):
            def ray_body(rr, c2):
                for u in range(ray_unroll):
                    do_ray(k, s, rr * ray_unroll + u)
                return c2
            lax.fori_loop(0, chunk // ray_unroll, ray_body, 0)

        for c_ in in_copies(0, 0):
            c_.start()

        def pair_body(k2, carry):
            for s in (0, 1):
                k = k2 * 2 + s

                @pl.when(k + 1 < n_chunks)
                def _():
                    for c_ in in_copies(k + 1, 1 - s):
                        c_.start()

                for c_ in in_copies(k, s):
                    c_.wait()

                @pl.when(k >= 2)
                def _():
                    out_copy(k - 2, s).wait()

                compute_chunk(k, s)
                out_copy(k, s).start()
            return carry

        lax.fori_loop(0, n_chunks // 2, pair_body, 0)
        out_copy(n_chunks - 2, 0).wait()
        out_copy(n_chunks - 1, 1).wait()

        pltpu.sync_copy(col_s, col_h.at[pl.ds(wid * col_rows, col_rows)])
        pltpu.sync_copy(dep_s, dep_h.at[pl.ds(wid * sca_rows, sca_rows)])
        pltpu.sync_copy(alp_s, alp_h.at[pl.ds(wid * sca_rows, sca_rows)])

    mesh = plsc.VectorSubcoreMesh(core_axis_name="c", subcore_axis_name="s")
    return pl.kernel(
        body,
        out_type=(
            jax.ShapeDtypeStruct((n_rays * 3 // 128, 128), f32),
            jax.ShapeDtypeStruct((n_rays // 128, 128), f32),
            jax.ShapeDtypeStruct((n_rays // 128, 128), f32),
            jax.ShapeDtypeStruct((n_rays, n_samples), f32),
        ),
        mesh=mesh,
        compiler_params=pltpu.CompilerParams(needs_layout_passes=False),
        scratch_types=[
            pltpu.VMEM((2, chunk, n_samples), f32),
            pltpu.VMEM((2, chunk, 3 * n_samples), f32),
            pltpu.VMEM((2, chunk, n_samples), f32),
            pltpu.VMEM((2, chunk, n_samples), f32),
            pltpu.VMEM((2, chunk, n_samples), f32),
            pltpu.VMEM((col_rows, 128), f32),
            pltpu.VMEM((sca_rows, 128), f32),
            pltpu.VMEM((sca_rows, 128), f32),
            pltpu.SemaphoreType.DMA((2,)),
            pltpu.SemaphoreType.DMA((2,)),
        ],
    )


@functools.partial(jax.jit, static_argnums=())
def kernel(sigma_vals, rgb_vals, z_vals, dists):
    n_rays, n_samples = sigma_vals.shape
    k = _make_kernel(n_rays, n_samples, chunk=64, ray_unroll=2)
    color, depth, alpha_coarse, weights = k(
        sigma_vals,
        rgb_vals.reshape(n_rays, 3 * n_samples),
        z_vals,
        dists,
    )
    return (color.reshape(n_rays, 3), depth.reshape(n_rays),
            alpha_coarse.reshape(n_rays), weights)
